# baseline (speedup 1.0000x reference)
import functools

import jax
import jax.numpy as jnp
from jax import lax
from jax.experimental import pallas as pl
from jax.experimental.pallas import tpu as pltpu
from jax.experimental.pallas import tpu_sc as plsc

_B = 16384
_H = 200
_D = 16
_NC = 2
_NS = 16
_NW = _NC * _NS
_CH = 2048
_NBC = _B // _CH
_UNITS = _H * _NBC
_PER_W = _UNITS // _NW
_TSZ = _CH * _D


def _make_gather():
    mesh = plsc.VectorSubcoreMesh(core_axis_name="c", subcore_axis_name="s")

    @functools.partial(
        pl.kernel,
        mesh=mesh,
        out_type=jax.ShapeDtypeStruct((_H, 2, _B * 8), jnp.float32),
        scratch_types=[
            pltpu.VMEM((_CH,), jnp.int32),
            pltpu.VMEM((_CH, _D), jnp.float32),
            pltpu.VMEM((_TSZ,), jnp.float32),
            pltpu.SemaphoreType.DMA,
        ],
        compiler_params=pltpu.CompilerParams(use_tc_tiling_on_sc=False, needs_layout_passes=False),
    )
    def k(xt_hbm, w_hbm, out_hbm, idx_v, rows_v, t_v, sem):
        wid = lax.axis_index("s") * _NC + lax.axis_index("c")
        lanes = lax.iota(jnp.int32, 16)
        pos0 = (lanes // 8) * (_CH * 8) + (lanes % 8) * 128
        pos_i = [pos0 + i for i in range(16)]
        _SPAN = _CH * 8 + 8 * 128 + 16

        def unit(j, _):
            u = wid * _PER_W + j
            h = u // _NBC
            bcb = u % _NBC
            pltpu.sync_copy(xt_hbm.at[h, pl.ds(bcb * _CH, _CH)], idx_v)
            pltpu.async_copy(w_hbm.at[idx_v], rows_v, sem).wait()

            def block(r0, _):
                base = (r0 // 8) * 1024 + (r0 % 8) * 16
                tv = t_v.at[pl.ds(base, _SPAN)]
                r = r0 * 16
                for i in range(16):
                    plsc.store_scatter(tv, [pos_i[i]], rows_v[r + i, :])
                return 0

            lax.fori_loop(0, _CH // 16, block, 0)
            for dh in range(2):
                pltpu.sync_copy(
                    t_v.at[pl.ds(dh * (_CH * 8), _CH * 8)],
                    out_hbm.at[h, dh, pl.ds(bcb * _CH * 8, _CH * 8)])
            return 0

        lax.fori_loop(0, _PER_W, unit, 0)

    return k


def kernel(x, W):
    x_t = jnp.swapaxes(x, 0, 1).astype(jnp.int32)
    out5 = _make_gather()(x_t, W)
    t = out5.reshape(_H, 2, _B // 128, 8, 128)
    t = t.transpose(0, 1, 3, 2, 4)
    t = t.reshape(_H, _D, _B)
    return t.transpose(2, 0, 1)


# speedup vs baseline: 1.1435x; 1.1435x over previous
import functools

import jax
import jax.numpy as jnp
from jax import lax
from jax.experimental import pallas as pl
from jax.experimental.pallas import tpu as pltpu
from jax.experimental.pallas import tpu_sc as plsc

_B = 16384
_H = 200
_D = 16
_NC = 2
_NS = 16
_NW = _NC * _NS
_CH = 2048
_NBC = _B // _CH
_UNITS = _H * _NBC
_PER_W = _UNITS // _NW
_TSZ = _CH * _D


def _make_gather():
    mesh = plsc.VectorSubcoreMesh(core_axis_name="c", subcore_axis_name="s")

    @functools.partial(
        pl.kernel,
        mesh=mesh,
        out_type=jax.ShapeDtypeStruct((_H, 2, _B * 8), jnp.float32),
        scratch_types=[
            pltpu.VMEM((_CH,), jnp.int32),
            pltpu.VMEM((_CH, _D), jnp.float32),
            pltpu.VMEM((_CH * (_D + 1),), jnp.float32),
            pltpu.VMEM((_TSZ,), jnp.float32),
            pltpu.SemaphoreType.DMA,
        ],
        compiler_params=pltpu.CompilerParams(use_tc_tiling_on_sc=False, needs_layout_passes=False),
    )
    def k(xt_hbm, w_hbm, out_hbm, idx_v, rows_v, rows_p, t_v, sem):
        wid = lax.axis_index("s") * _NC + lax.axis_index("c")
        lanes = lax.iota(jnp.int32, 16)
        pos17 = lanes * (_D + 1)

        def unit(j, _):
            u = wid * _PER_W + j
            h = u // _NBC
            bcb = u % _NBC
            pltpu.sync_copy(xt_hbm.at[h, pl.ds(bcb * _CH, _CH)], idx_v)
            pltpu.async_copy(w_hbm.at[idx_v], rows_v, sem).wait()

            def repack(r0, _):
                r = r0 * 16
                p = r * (_D + 1)
                for i in range(16):
                    rows_p[pl.ds(p + i * (_D + 1), _D)] = rows_v[r + i, :]
                return 0

            lax.fori_loop(0, _CH // 16, repack, 0)

            def block(bcl, _):
                sb = bcl * (128 * (_D + 1))
                obase = bcl * 1024
                for d in range(16):
                    toff = (d // 8) * (_CH * 8) + (d % 8) * 128
                    for bi in range(8):
                        src = pos17 + (sb + bi * 16 * (_D + 1) + d)
                        v = plsc.load_gather(rows_p, [src])
                        t_v[pl.ds(obase + toff + bi * 16, 16)] = v
                return 0

            lax.fori_loop(0, _CH // 128, block, 0)
            for dh in range(2):
                pltpu.sync_copy(
                    t_v.at[pl.ds(dh * (_CH * 8), _CH * 8)],
                    out_hbm.at[h, dh, pl.ds(bcb * _CH * 8, _CH * 8)])
            return 0

        lax.fori_loop(0, _PER_W, unit, 0)

    return k


def kernel(x, W):
    x_t = jnp.swapaxes(x, 0, 1).astype(jnp.int32)
    out5 = _make_gather()(x_t, W)
    t = out5.reshape(_H, 2, _B // 128, 8, 128)
    t = t.transpose(0, 1, 3, 2, 4)
    t = t.reshape(_H, _D, _B)
    return t.transpose(2, 0, 1)
